# baseline (speedup 1.0000x reference)
import jax
import jax.numpy as jnp
from jax import lax
from jax.experimental import pallas as pl
from jax.experimental.pallas import tpu as pltpu
from jax.experimental.pallas import tpu_sc as plsc

NC = 2
NS = 16
B = 128
HW = 64
WD = 16


def _sc_aggregate(table2, srcs, dsts, zeros_col, zeros_deg, ones_row,
                  n_rows, g_chunks, with_deg):
  rpt = n_rows // NS

  out_type = [jax.ShapeDtypeStruct((NC * n_rows, HW), jnp.bfloat16)]
  scratch = [
      pltpu.VMEM((g_chunks + 1, B), jnp.int32),
      pltpu.VMEM((g_chunks + 1, B), jnp.int32),
      pltpu.VMEM((B, HW), jnp.bfloat16),
      pltpu.VMEM((B, HW), jnp.bfloat16),
      pltpu.VMEM_SHARED((n_rows, HW), jnp.bfloat16),
      pltpu.SemaphoreType.DMA,
      pltpu.SemaphoreType.DMA,
  ]
  if with_deg:
    out_type.append(jax.ShapeDtypeStruct((NC * n_rows, WD), jnp.float32))
    scratch += [
        pltpu.VMEM((B, WD), jnp.float32),
        pltpu.VMEM_SHARED((n_rows, WD), jnp.float32),
    ]

  mesh = plsc.VectorSubcoreMesh(core_axis_name="c", subcore_axis_name="s")

  def body(table_hbm, srcs_hbm, dsts_hbm, zc_hbm, zd_hbm, ones_hbm,
           part_hbm, *rest):
    if with_deg:
      degp_hbm, idx_src, idx_dst, buf0, buf1, acc, sem0, sem1, ones_v, dacc \
          = rest
    else:
      idx_src, idx_dst, buf0, buf1, acc, sem0, sem1 = rest
    c = lax.axis_index("c")
    s = lax.axis_index("s")
    r0 = s * rpt
    my_table = table_hbm.at[c]
    pltpu.sync_copy(zc_hbm.at[pl.ds(r0, rpt)], acc.at[pl.ds(r0, rpt)])
    if with_deg:
      pltpu.sync_copy(zd_hbm.at[pl.ds(r0, rpt)], dacc.at[pl.ds(r0, rpt)])
      pltpu.sync_copy(ones_hbm, ones_v)
    pltpu.sync_copy(srcs_hbm.at[s], idx_src.at[pl.ds(0, g_chunks)])
    pltpu.sync_copy(dsts_hbm.at[s], idx_dst.at[pl.ds(0, g_chunks)])
    zero16 = jnp.zeros((16,), jnp.int32)
    for j in range(B // 16):
      idx_src[g_chunks, pl.ds(16 * j, 16)] = zero16
    plsc.subcore_barrier()

    def start_gather(g, buf, sem):
      pltpu.async_copy(my_table.at[idx_src.at[g]], buf, sem)

    def finish_chunk(g, parity, buf, sem):
      pltpu.make_async_copy(my_table.at[idx_src.at[g]], buf, sem).wait()
      pltpu.sync_copy(buf, acc.at[idx_dst.at[g]], add=True)
      if with_deg:
        @pl.when(c == parity)
        def _():
          pltpu.sync_copy(ones_v, dacc.at[idx_dst.at[g]], add=True)

    start_gather(0, buf0, sem0)

    def loop_body(g, carry):
      start_gather(2 * g + 1, buf1, sem1)
      finish_chunk(2 * g, 0, buf0, sem0)
      start_gather(2 * g + 2, buf0, sem0)
      finish_chunk(2 * g + 1, 1, buf1, sem1)
      return carry

    lax.fori_loop(0, g_chunks // 2, loop_body, 0)
    pltpu.make_async_copy(my_table.at[idx_src.at[g_chunks]], buf0,
                          sem0).wait()
    plsc.subcore_barrier()
    pltpu.sync_copy(acc.at[pl.ds(r0, rpt)],
                    part_hbm.at[pl.ds(c * n_rows + r0, rpt)])
    if with_deg:
      pltpu.sync_copy(dacc.at[pl.ds(r0, rpt)],
                      degp_hbm.at[pl.ds(c * n_rows + r0, rpt)])

  run = pl.kernel(body, out_type=tuple(out_type), mesh=mesh,
                  scratch_types=tuple(scratch),
                  compiler_params=pltpu.CompilerParams(
                      use_tc_tiling_on_sc=False))
  return run(table2, srcs, dsts, zeros_col, zeros_deg, ones_row)


def _tc_layer1(x_ref, p1_ref, dg_ref, ws1_ref, wn1_ref, b1_ref,
               wn2_ref, ws2_ref, b2_ref, z2_ref, s2_ref):
  p = p1_ref[...].astype(jnp.float32)
  d3 = dg_ref[...]
  deg = (d3[0] + d3[1])[:, 0:1]
  inv = 1.0 / jnp.maximum(deg, 1.0)
  hn = jnp.concatenate([p[0], p[1]], axis=1) * inv
  x = x_ref[...]
  def matT(a, w_ref):
    return lax.dot_general(a, w_ref[...], (((1,), (1,)), ((), ())),
                           preferred_element_type=jnp.float32)

  h1 = jnp.maximum(matT(x, ws1_ref) + matT(hn, wn1_ref) + b1_ref[...], 0.0)
  z = matT(h1, wn2_ref).astype(jnp.bfloat16)
  z2_ref[0] = z[:, :HW]
  z2_ref[1] = z[:, HW:]
  s2_ref[...] = matT(h1, ws2_ref) + b2_ref[...]


def _tc_layer2(s2_ref, p2_ref, dg_ref, out_ref):
  p = p2_ref[...].astype(jnp.float32)
  d3 = dg_ref[...]
  deg = (d3[0] + d3[1])[:, 0:1]
  inv = 1.0 / jnp.maximum(deg, 1.0)
  out_ref[...] = s2_ref[...] + jnp.concatenate([p[0], p[1]], axis=1) * inv


def kernel(features, edge_index, W_self1, W_neigh1, b1, W_self2, W_neigh2, b2):
  n, d = features.shape
  h = W_self1.shape[0]
  e = edge_index.shape[1]

  per_w = -(-e // NS)
  g_chunks = 2 * (-(-per_w // (2 * B)))
  e_pad = NS * g_chunks * B
  n_rows = ((n + 1 + NS * 8 - 1) // (NS * 8)) * (NS * 8)

  src = edge_index[0]
  dst = edge_index[1]
  pad = e_pad - e
  srcs = jnp.concatenate([src, jnp.zeros((pad,), src.dtype)])
  dsts = jnp.concatenate([dst, jnp.full((pad,), n, dst.dtype)])
  srcs = srcs.reshape(NS, g_chunks, B).astype(jnp.int32)
  dsts = dsts.reshape(NS, g_chunks, B).astype(jnp.int32)

  zeros_col = jnp.zeros((n_rows, HW), jnp.bfloat16)
  zeros_deg = jnp.zeros((n_rows, WD), jnp.float32)
  ones_row = jnp.ones((B, WD), jnp.float32)

  f2 = jnp.stack([features[:, :HW], features[:, HW:]]).astype(jnp.bfloat16)

  part1, degp = _sc_aggregate(f2, srcs, dsts, zeros_col, zeros_deg,
                              ones_row, n_rows, g_chunks, with_deg=True)
  part1 = part1.reshape(NC, n_rows, HW)
  degp = degp.reshape(NC, n_rows, WD)

  rb = 1000
  grid = (n // rb,)
  z2, s2 = pl.pallas_call(
      _tc_layer1,
      grid=grid,
      in_specs=[
          pl.BlockSpec((rb, d), lambda i: (i, 0)),
          pl.BlockSpec((NC, rb, HW), lambda i: (0, i, 0)),
          pl.BlockSpec((NC, rb, WD), lambda i: (0, i, 0)),
          pl.BlockSpec((h, d), lambda i: (0, 0)),
          pl.BlockSpec((h, d), lambda i: (0, 0)),
          pl.BlockSpec((1, h), lambda i: (0, 0)),
          pl.BlockSpec((d, h), lambda i: (0, 0)),
          pl.BlockSpec((d, h), lambda i: (0, 0)),
          pl.BlockSpec((1, d), lambda i: (0, 0)),
      ],
      out_specs=[
          pl.BlockSpec((NC, rb, HW), lambda i: (0, i, 0)),
          pl.BlockSpec((rb, d), lambda i: (i, 0)),
      ],
      out_shape=[
          jax.ShapeDtypeStruct((NC, n, HW), jnp.bfloat16),
          jax.ShapeDtypeStruct((n, d), jnp.float32),
      ],
  )(features, part1, degp, W_self1, W_neigh1, b1.reshape(1, h),
    W_neigh2, W_self2, b2.reshape(1, d))

  part2 = _sc_aggregate(z2, srcs, dsts, zeros_col, zeros_deg, ones_row,
                        n_rows, g_chunks, with_deg=False)[0]
  part2 = part2.reshape(NC, n_rows, HW)

  out = pl.pallas_call(
      _tc_layer2,
      grid=grid,
      in_specs=[
          pl.BlockSpec((rb, d), lambda i: (i, 0)),
          pl.BlockSpec((NC, rb, HW), lambda i: (0, i, 0)),
          pl.BlockSpec((NC, rb, WD), lambda i: (0, i, 0)),
      ],
      out_specs=pl.BlockSpec((rb, d), lambda i: (i, 0)),
      out_shape=jax.ShapeDtypeStruct((n, d), jnp.float32),
  )(s2, part2, degp)
  return out
